# baseline (speedup 1.0000x reference)
import functools

import jax
import jax.numpy as jnp
from jax import lax
from jax.experimental import pallas as pl
from jax.experimental.pallas import tpu as pltpu
from jax.experimental.pallas import tpu_sc as plsc

N = 10000
E = 320000
D = 128
DE = 16
DH = 128
G = 64
DEPTH = 3

NC = 2
NS = 16
NW = NC * NS
LANES = 16
CHUNK = 64
NCHUNKS = E // CHUNK
NQUADS = NCHUNKS // 4
RPT = N // NS


def _make_edge_pass(write_h0: bool):
    mesh = plsc.VectorSubcoreMesh(core_axis_name="c", subcore_axis_name="s")

    out_type = [jax.ShapeDtypeStruct((NC, NS, RPT, DH), jnp.float32)]
    if write_h0:
        out_type = [jax.ShapeDtypeStruct((E, DH), jnp.float32)] + out_type

    scratch_types = [
        pltpu.VMEM((CHUNK,), jnp.int32),
        pltpu.VMEM((CHUNK,), jnp.int32),
        pltpu.VMEM((CHUNK,), jnp.int32),
        pltpu.VMEM((CHUNK,), jnp.int32),
        pltpu.VMEM((2 * CHUNK,), jnp.int32),
        pltpu.VMEM((2 * CHUNK,), jnp.int32),
        pltpu.VMEM((CHUNK, DH), jnp.float32),
        pltpu.VMEM((CHUNK, DH), jnp.float32),
        pltpu.VMEM((CHUNK, DH), jnp.float32),
        pltpu.VMEM((CHUNK, DH), jnp.float32),
        pltpu.VMEM((2 * CHUNK, DH), jnp.float32),
        pltpu.VMEM_SHARED((N, DH), jnp.float32),
        pltpu.SemaphoreType.DMA,
        pltpu.SemaphoreType.DMA,
    ]

    def body(base_hbm, table_hbm, src_hbm, dst_hbm, *rest):
        if write_h0:
            h0_out, spart, *rest = rest
        else:
            spart, *rest = rest
        (src_a0, src_a1, src_b0, src_b1, dstp_0, dstp_1,
         rows_a, rows_b, base_a, base_b, outp, acc, sem_a, sem_b) = rest
        cid = lax.axis_index("c")
        sid = lax.axis_index("s")
        wid = sid * NC + cid

        def zrow(r, _):
            for sl in range(DH // LANES):
                rows_a[r, pl.ds(sl * LANES, LANES)] = jnp.zeros((LANES,), jnp.float32)
            return _
        lax.fori_loop(0, CHUNK, zrow, None)
        row0 = sid * RPT
        off = 0
        while off < RPT:
            nr = min(CHUNK, RPT - off)
            pltpu.sync_copy(rows_a.at[pl.ds(0, nr)], acc.at[pl.ds(row0 + off, nr)])
            off += nr
        plsc.subcore_barrier()

        qlo = (wid * NQUADS) // NW
        qhi = ((wid + 1) * NQUADS) // NW
        clo = 4 * qlo
        nq = qhi - qlo

        def clamp(c):
            return jnp.minimum(c, NCHUNKS - 1)

        def clamp_pair(c):
            return jnp.minimum(c, NCHUNKS - 2)

        def issue_a(c, src_v, dstp_v, pf_c, pf_src_v):
            c = clamp_pair(c)
            pltpu.async_copy(table_hbm.at[src_v], rows_a, sem_a)
            pltpu.async_copy(base_hbm.at[pl.ds(c * CHUNK, CHUNK)], base_a, sem_a)
            pltpu.async_copy(dst_hbm.at[pl.ds(c * CHUNK, 2 * CHUNK)], dstp_v, sem_a)
            pltpu.async_copy(src_hbm.at[pl.ds(clamp(pf_c) * CHUNK, CHUNK)],
                             pf_src_v, sem_a)

        def issue_b(c, src_v, pf_c, pf_src_v):
            c = clamp(c)
            pltpu.async_copy(table_hbm.at[src_v], rows_b, sem_b)
            pltpu.async_copy(base_hbm.at[pl.ds(c * CHUNK, CHUNK)], base_b, sem_b)
            pltpu.async_copy(src_hbm.at[pl.ds(clamp(pf_c) * CHUNK, CHUNK)],
                             pf_src_v, sem_b)

        def drain_a(dstp_v, pf_src_v):
            pltpu.make_async_copy(base_hbm.at[pl.ds(0, CHUNK)], rows_a, sem_a).wait()
            pltpu.make_async_copy(base_hbm.at[pl.ds(0, CHUNK)], base_a, sem_a).wait()
            pltpu.make_async_copy(dst_hbm.at[pl.ds(0, 2 * CHUNK)], dstp_v, sem_a).wait()
            pltpu.make_async_copy(src_hbm.at[pl.ds(0, CHUNK)], pf_src_v, sem_a).wait()

        def drain_b(pf_src_v):
            pltpu.make_async_copy(base_hbm.at[pl.ds(0, CHUNK)], rows_b, sem_b).wait()
            pltpu.make_async_copy(base_hbm.at[pl.ds(0, CHUNK)], base_b, sem_b).wait()
            pltpu.make_async_copy(src_hbm.at[pl.ds(0, CHUNK)], pf_src_v, sem_b).wait()

        def relu_add(rows_v, base_v, out_off):
            @plsc.parallel_loop(0, CHUNK, unroll=4)
            def _row(r):
                for sl in range(DH // LANES):
                    col = pl.ds(sl * LANES, LANES)
                    outp[out_off + r, col] = jnp.maximum(
                        rows_v[r, col] + base_v[r, col], 0.0)

        def finish_a(dstp_v, pf_src_v):
            drain_a(dstp_v, pf_src_v)
            relu_add(rows_a, base_a, 0)

        def finish_b(c, dstp_v, pf_src_v):
            c = clamp(c)
            drain_b(pf_src_v)
            relu_add(rows_b, base_b, CHUNK)
            if write_h0:
                pltpu.sync_copy(outp, h0_out.at[pl.ds((c - 1) * CHUNK, 2 * CHUNK)])
            pltpu.sync_copy(outp, acc.at[dstp_v], add=True)

        pltpu.sync_copy(src_hbm.at[pl.ds(clo * CHUNK, CHUNK)], src_a0)
        pltpu.sync_copy(src_hbm.at[pl.ds(clamp(clo + 1) * CHUNK, CHUNK)], src_b0)
        issue_a(clo, src_a0, dstp_0, clo + 2, src_a1)

        def quad(t, _):
            qb = clo + 4 * t
            issue_b(qb + 1, src_b0, qb + 3, src_b1)
            finish_a(dstp_0, src_a1)
            issue_a(qb + 2, src_a1, dstp_1, qb + 4, src_a0)
            finish_b(qb + 1, dstp_0, src_b1)
            issue_b(qb + 3, src_b1, qb + 5, src_b0)
            finish_a(dstp_1, src_a0)
            issue_a(qb + 4, src_a0, dstp_0, qb + 6, src_a1)
            finish_b(qb + 3, dstp_1, src_b0)
            return _

        lax.fori_loop(0, nq, quad, None)
        drain_a(dstp_0, src_a1)

        plsc.subcore_barrier()
        pltpu.sync_copy(acc.at[pl.ds(row0, RPT)], spart.at[cid, sid])

    return pl.kernel(body, out_type=tuple(out_type), mesh=mesh,
                     scratch_types=scratch_types)


_encoder_pass = _make_edge_pass(write_h0=True)
_layer_pass = _make_edge_pass(write_h0=False)


def _u_body(x_ref, w_ref, o_ref):
    o_ref[...] = jnp.dot(x_ref[...], w_ref[...], preferred_element_type=jnp.float32)


def _u_kernel(x, w1):
    return pl.pallas_call(
        _u_body, out_shape=jax.ShapeDtypeStruct((N, DH), jnp.float32))(x, w1)


def _ea_body(a_ref, w_ref, b_ref, o_ref):
    o_ref[...] = (jnp.dot(a_ref[...], w_ref[...], preferred_element_type=jnp.float32)
                  + b_ref[...])


def _ea_kernel(edge_attr, w2, b2):
    BE = 8000
    return pl.pallas_call(
        _ea_body,
        grid=(E // BE,),
        in_specs=[pl.BlockSpec((BE, DE), lambda i: (i, 0)),
                  pl.BlockSpec((DE, DH), lambda i: (0, 0)),
                  pl.BlockSpec((1, DH), lambda i: (0, 0))],
        out_specs=pl.BlockSpec((BE, DH), lambda i: (i, 0)),
        out_shape=jax.ShapeDtypeStruct((E, DH), jnp.float32),
    )(edge_attr, w2, b2)


def _t_body(sp_ref, w_ref, b_ref, o_ref):
    s = sp_ref[0] + sp_ref[1]
    o_ref[...] = (jnp.dot(s, w_ref[...], preferred_element_type=jnp.float32)
                  + b_ref[...])


def _t_kernel(spart, w, b2):
    return pl.pallas_call(
        _t_body, out_shape=jax.ShapeDtypeStruct((N, DH), jnp.float32))(spart, w, b2)


def _final_body(x_ref, sp_ref, bv_ref, w1_ref, w2_ref, be_ref, wh_ref, bh_ref, o_ref):
    s = sp_ref[0] + sp_ref[1]
    xn = (jnp.dot(x_ref[...], w1_ref[...], preferred_element_type=jnp.float32)
          + jnp.dot(s, w2_ref[...], preferred_element_type=jnp.float32)
          + be_ref[...])
    xn = jnp.maximum(xn, 0.0)
    seg = lax.broadcasted_iota(jnp.int32, (N, G), 1)
    onehot = (bv_ref[...] == seg).astype(jnp.float32)
    pooled = lax.dot_general(onehot, xn, (((0,), (0,)), ((), ())),
                             preferred_element_type=jnp.float32)
    o_ref[...] = (jnp.dot(pooled, wh_ref[...], preferred_element_type=jnp.float32)
                  + bh_ref[...])


def _final_kernel(x, spart, bv2, w1, w2, be2, wh, bh2):
    return pl.pallas_call(
        _final_body, out_shape=jax.ShapeDtypeStruct((G, 1), jnp.float32),
    )(x, spart, bv2, w1, w2, be2, wh, bh2)


@jax.jit
def kernel(x, edge_index, edge_attr, batch_vec, W_enc, b_enc, W_layer, b_layer,
           W_e2n, b_e2n, W_head, b_head):
    src = edge_index[0].astype(jnp.int32)
    dst = edge_index[1].astype(jnp.int32)

    u = _u_kernel(x, W_enc[:D])
    ea = _ea_kernel(edge_attr, W_enc[D:], b_enc.reshape(1, DH))

    h0, spart = _encoder_pass(ea, u, src, dst)
    spart = spart.reshape(NC, N, DH)
    b_layer2 = b_layer.reshape(1, DH)
    for _ in range(DEPTH):
        t = _t_kernel(spart, W_layer, b_layer2)
        (spart,) = _layer_pass(h0, t, src, dst)
        spart = spart.reshape(NC, N, DH)

    return _final_kernel(x, spart, batch_vec.astype(jnp.int32).reshape(N, 1),
                         W_e2n[:D], W_e2n[D:], b_e2n.reshape(1, DH),
                         W_head, b_head.reshape(1, 1))


# speedup vs baseline: 1.0675x; 1.0675x over previous
import functools

import jax
import jax.numpy as jnp
from jax import lax
from jax.experimental import pallas as pl
from jax.experimental.pallas import tpu as pltpu
from jax.experimental.pallas import tpu_sc as plsc

N = 10000
E = 320000
D = 128
DE = 16
DH = 128
G = 64
DEPTH = 3

NC = 2
NS = 16
NW = NC * NS
LANES = 16
CHUNK = 80
NCHUNKS = E // CHUNK
NQUADS = NCHUNKS // 4
RPT = N // NS


def _make_edge_pass(write_h0: bool):
    mesh = plsc.VectorSubcoreMesh(core_axis_name="c", subcore_axis_name="s")

    out_type = [jax.ShapeDtypeStruct((NC, NS, RPT, DH), jnp.float32)]
    if write_h0:
        out_type = [jax.ShapeDtypeStruct((E, DH), jnp.float32)] + out_type

    scratch_types = [
        pltpu.VMEM((CHUNK,), jnp.int32),
        pltpu.VMEM((CHUNK,), jnp.int32),
        pltpu.VMEM((CHUNK,), jnp.int32),
        pltpu.VMEM((CHUNK,), jnp.int32),
        pltpu.VMEM((CHUNK,), jnp.int32),
        pltpu.VMEM((CHUNK,), jnp.int32),
        pltpu.VMEM((CHUNK, DH), jnp.float32),
        pltpu.VMEM((CHUNK, DH), jnp.float32),
        pltpu.VMEM((CHUNK, DH), jnp.float32),
        pltpu.VMEM((CHUNK, DH), jnp.float32),
        pltpu.VMEM_SHARED((N, DH), jnp.float32),
        pltpu.SemaphoreType.DMA,
        pltpu.SemaphoreType.DMA,
        pltpu.SemaphoreType.DMA,
        pltpu.SemaphoreType.DMA,
    ]

    def body(base_hbm, table_hbm, src_hbm, dst_hbm, *rest):
        if write_h0:
            h0_out, spart, *rest = rest
        else:
            spart, *rest = rest
        (src_a0, src_a1, src_b0, src_b1, dst_a, dst_b,
         rows_a, rows_b, base_a, base_b, acc,
         sem_a, sem_b, sem_wa, sem_wb) = rest
        cid = lax.axis_index("c")
        sid = lax.axis_index("s")
        wid = sid * NC + cid

        def zrow(r, _):
            for sl in range(DH // LANES):
                rows_a[r, pl.ds(sl * LANES, LANES)] = jnp.zeros((LANES,), jnp.float32)
            return _
        lax.fori_loop(0, CHUNK, zrow, None)
        row0 = sid * RPT
        off = 0
        while off < RPT:
            nr = min(CHUNK, RPT - off)
            pltpu.sync_copy(rows_a.at[pl.ds(0, nr)], acc.at[pl.ds(row0 + off, nr)])
            off += nr
        plsc.subcore_barrier()

        qlo = (wid * NQUADS) // NW
        qhi = ((wid + 1) * NQUADS) // NW
        clo = 4 * qlo
        nq = qhi - qlo

        def clamp(c):
            return jnp.minimum(c, NCHUNKS - 1)

        def issue(c, rows_v, base_v, dst_v, src_v, pf_c, pf_src_v, sem,
                  sem_w, wdrain):
            if write_h0 and wdrain:
                pltpu.make_async_copy(base_hbm.at[pl.ds(0, CHUNK)], rows_v,
                                      sem_w).wait()
            c = clamp(c)
            pltpu.async_copy(table_hbm.at[src_v], rows_v, sem)
            pltpu.async_copy(base_hbm.at[pl.ds(c * CHUNK, CHUNK)], base_v, sem)
            pltpu.async_copy(dst_hbm.at[pl.ds(c * CHUNK, CHUNK)], dst_v, sem)
            pltpu.async_copy(src_hbm.at[pl.ds(clamp(pf_c) * CHUNK, CHUNK)],
                             pf_src_v, sem)

        def drain(rows_v, base_v, dst_v, pf_src_v, sem):
            pltpu.make_async_copy(base_hbm.at[pl.ds(0, CHUNK)], rows_v, sem).wait()
            pltpu.make_async_copy(base_hbm.at[pl.ds(0, CHUNK)], base_v, sem).wait()
            pltpu.make_async_copy(dst_hbm.at[pl.ds(0, CHUNK)], dst_v, sem).wait()
            pltpu.make_async_copy(src_hbm.at[pl.ds(0, CHUNK)], pf_src_v, sem).wait()

        def finish(c, rows_v, base_v, dst_v, pf_src_v, sem, sem_w):
            c = clamp(c)
            drain(rows_v, base_v, dst_v, pf_src_v, sem)
            @plsc.parallel_loop(0, CHUNK, unroll=8)
            def _row(r):
                for sl in range(DH // LANES):
                    col = pl.ds(sl * LANES, LANES)
                    rows_v[r, col] = jnp.maximum(rows_v[r, col] + base_v[r, col], 0.0)
            if write_h0:
                pltpu.async_copy(rows_v, h0_out.at[pl.ds(c * CHUNK, CHUNK)], sem_w)
            pltpu.sync_copy(rows_v, acc.at[dst_v], add=True)

        pltpu.sync_copy(src_hbm.at[pl.ds(clo * CHUNK, CHUNK)], src_a0)
        pltpu.sync_copy(src_hbm.at[pl.ds(clamp(clo + 1) * CHUNK, CHUNK)], src_b0)
        issue(clo, rows_a, base_a, dst_a, src_a0, clo + 2, src_a1, sem_a,
              sem_wa, False)

        def quad(qb, wd_b0):
            issue(qb + 1, rows_b, base_b, dst_b, src_b0, qb + 3, src_b1,
                  sem_b, sem_wb, wd_b0)
            finish(qb, rows_a, base_a, dst_a, src_a1, sem_a, sem_wa)
            issue(qb + 2, rows_a, base_a, dst_a, src_a1, qb + 4, src_a0,
                  sem_a, sem_wa, True)
            finish(qb + 1, rows_b, base_b, dst_b, src_b1, sem_b, sem_wb)
            issue(qb + 3, rows_b, base_b, dst_b, src_b1, qb + 5, src_b0,
                  sem_b, sem_wb, True)
            finish(qb + 2, rows_a, base_a, dst_a, src_a0, sem_a, sem_wa)
            issue(qb + 4, rows_a, base_a, dst_a, src_a0, qb + 6, src_a1,
                  sem_a, sem_wa, True)
            finish(qb + 3, rows_b, base_b, dst_b, src_b0, sem_b, sem_wb)

        quad(clo, False)

        def quad_loop(t, _):
            quad(clo + 4 * t, True)
            return _

        lax.fori_loop(1, nq, quad_loop, None)
        drain(rows_a, base_a, dst_a, src_a1, sem_a)
        if write_h0:
            pltpu.make_async_copy(base_hbm.at[pl.ds(0, CHUNK)], rows_b, sem_wb).wait()

        plsc.subcore_barrier()
        pltpu.sync_copy(acc.at[pl.ds(row0, RPT)], spart.at[cid, sid])

    return pl.kernel(body, out_type=tuple(out_type), mesh=mesh,
                     scratch_types=scratch_types)


_encoder_pass = _make_edge_pass(write_h0=True)
_layer_pass = _make_edge_pass(write_h0=False)


def _enc_body(a_ref, w2_ref, b_ref, x_ref, w1_ref, ea_ref, u_ref):
    ea_ref[...] = (jnp.dot(a_ref[...], w2_ref[...],
                           preferred_element_type=jnp.float32) + b_ref[...])
    @pl.when(pl.program_id(0) == 0)
    def _():
        u_ref[...] = jnp.dot(x_ref[...], w1_ref[...],
                             preferred_element_type=jnp.float32)


def _enc_kernel(edge_attr, w2, b2, x, w1):
    BE = 8000
    return pl.pallas_call(
        _enc_body,
        grid=(E // BE,),
        in_specs=[pl.BlockSpec((BE, DE), lambda i: (i, 0)),
                  pl.BlockSpec((DE, DH), lambda i: (0, 0)),
                  pl.BlockSpec((1, DH), lambda i: (0, 0)),
                  pl.BlockSpec((N, D), lambda i: (0, 0)),
                  pl.BlockSpec((D, DH), lambda i: (0, 0))],
        out_specs=[pl.BlockSpec((BE, DH), lambda i: (i, 0)),
                   pl.BlockSpec((N, DH), lambda i: (0, 0))],
        out_shape=[jax.ShapeDtypeStruct((E, DH), jnp.float32),
                   jax.ShapeDtypeStruct((N, DH), jnp.float32)],
    )(edge_attr, w2, b2, x, w1)


def _t_body(sp_ref, w_ref, b_ref, o_ref):
    s = sp_ref[0] + sp_ref[1]
    o_ref[...] = (jnp.dot(s, w_ref[...], preferred_element_type=jnp.float32)
                  + b_ref[...])


def _t_kernel(spart, w, b2):
    return pl.pallas_call(
        _t_body, out_shape=jax.ShapeDtypeStruct((N, DH), jnp.float32))(spart, w, b2)


def _final_body(x_ref, sp_ref, bv_ref, w1_ref, w2_ref, be_ref, wh_ref, bh_ref, o_ref):
    s = sp_ref[0] + sp_ref[1]
    xn = (jnp.dot(x_ref[...], w1_ref[...], preferred_element_type=jnp.float32)
          + jnp.dot(s, w2_ref[...], preferred_element_type=jnp.float32)
          + be_ref[...])
    xn = jnp.maximum(xn, 0.0)
    seg = lax.broadcasted_iota(jnp.int32, (N, G), 1)
    onehot = (bv_ref[...] == seg).astype(jnp.float32)
    pooled = lax.dot_general(onehot, xn, (((0,), (0,)), ((), ())),
                             preferred_element_type=jnp.float32)
    o_ref[...] = (jnp.dot(pooled, wh_ref[...], preferred_element_type=jnp.float32)
                  + bh_ref[...])


def _final_kernel(x, spart, bv2, w1, w2, be2, wh, bh2):
    return pl.pallas_call(
        _final_body, out_shape=jax.ShapeDtypeStruct((G, 1), jnp.float32),
    )(x, spart, bv2, w1, w2, be2, wh, bh2)


@jax.jit
def kernel(x, edge_index, edge_attr, batch_vec, W_enc, b_enc, W_layer, b_layer,
           W_e2n, b_e2n, W_head, b_head):
    src = edge_index[0].astype(jnp.int32)
    dst = edge_index[1].astype(jnp.int32)

    ea, u = _enc_kernel(edge_attr, W_enc[D:], b_enc.reshape(1, DH), x, W_enc[:D])

    h0, spart = _encoder_pass(ea, u, src, dst)
    spart = spart.reshape(NC, N, DH)
    b_layer2 = b_layer.reshape(1, DH)
    for _ in range(DEPTH):
        t = _t_kernel(spart, W_layer, b_layer2)
        (spart,) = _layer_pass(h0, t, src, dst)
        spart = spart.reshape(NC, N, DH)

    return _final_kernel(x, spart, batch_vec.astype(jnp.int32).reshape(N, 1),
                         W_e2n[:D], W_e2n[D:], b_e2n.reshape(1, DH),
                         W_head, b_head.reshape(1, 1))


# speedup vs baseline: 1.1321x; 1.0605x over previous
import functools

import jax
import jax.numpy as jnp
from jax import lax
from jax.experimental import pallas as pl
from jax.experimental.pallas import tpu as pltpu
from jax.experimental.pallas import tpu_sc as plsc

N = 10000
E = 320000
D = 128
DE = 16
DH = 128
G = 64
DEPTH = 3

NC = 2
NS = 16
NW = NC * NS
LANES = 16
CHUNK = 80
NCHUNKS = E // CHUNK
NQUADS = NCHUNKS // 4
RPT = N // NS


def _make_edge_pass(write_h0: bool):
    mesh = plsc.VectorSubcoreMesh(core_axis_name="c", subcore_axis_name="s")

    out_type = [jax.ShapeDtypeStruct((NC, NS, RPT, DH), jnp.float32)]
    if write_h0:
        out_type = [jax.ShapeDtypeStruct((E, DH), jnp.float32)] + out_type

    scratch_types = [
        pltpu.VMEM((CHUNK,), jnp.int32),
        pltpu.VMEM((CHUNK,), jnp.int32),
        pltpu.VMEM((CHUNK,), jnp.int32),
        pltpu.VMEM((CHUNK,), jnp.int32),
        pltpu.VMEM((CHUNK,), jnp.int32),
        pltpu.VMEM((CHUNK,), jnp.int32),
        pltpu.VMEM((CHUNK, DH), jnp.float32),
        pltpu.VMEM((CHUNK, DH), jnp.float32),
        pltpu.VMEM((CHUNK, DH), jnp.float32),
        pltpu.VMEM((CHUNK, DH), jnp.float32),
        pltpu.VMEM_SHARED((N, DH), jnp.float32),
        pltpu.SemaphoreType.DMA,
        pltpu.SemaphoreType.DMA,
        pltpu.SemaphoreType.DMA,
        pltpu.SemaphoreType.DMA,
    ]

    def body(base_hbm, table_hbm, src_hbm, dst_hbm, *rest):
        if write_h0:
            h0_out, spart, *rest = rest
        else:
            spart, *rest = rest
        (src_a0, src_a1, src_b0, src_b1, dst_a, dst_b,
         rows_a, rows_b, base_a, base_b, acc,
         sem_a, sem_b, sem_wa, sem_wb) = rest
        cid = lax.axis_index("c")
        sid = lax.axis_index("s")
        wid = sid * NC + cid

        def zrow(r, _):
            for sl in range(DH // LANES):
                rows_a[r, pl.ds(sl * LANES, LANES)] = jnp.zeros((LANES,), jnp.float32)
            return _
        lax.fori_loop(0, CHUNK, zrow, None)
        row0 = sid * RPT
        off = 0
        while off < RPT:
            nr = min(CHUNK, RPT - off)
            pltpu.sync_copy(rows_a.at[pl.ds(0, nr)], acc.at[pl.ds(row0 + off, nr)])
            off += nr
        plsc.subcore_barrier()

        qlo = (wid * NQUADS) // NW
        qhi = ((wid + 1) * NQUADS) // NW
        clo = 4 * qlo
        nq = qhi - qlo

        def clamp(c):
            return jnp.minimum(c, NCHUNKS - 1)

        def issue(c, rows_v, base_v, dst_v, src_v, pf_c, pf_src_v, sem,
                  sem_w, wdrain):
            if write_h0 and wdrain:
                pltpu.make_async_copy(base_hbm.at[pl.ds(0, CHUNK)], rows_v,
                                      sem_w).wait()
            c = clamp(c)
            pltpu.async_copy(table_hbm.at[src_v], rows_v, sem)
            pltpu.async_copy(base_hbm.at[pl.ds(c * CHUNK, CHUNK)], base_v, sem)
            pltpu.async_copy(dst_hbm.at[pl.ds(c * CHUNK, CHUNK)], dst_v, sem)
            pltpu.async_copy(src_hbm.at[pl.ds(clamp(pf_c) * CHUNK, CHUNK)],
                             pf_src_v, sem)

        def drain(rows_v, base_v, dst_v, pf_src_v, sem):
            pltpu.make_async_copy(base_hbm.at[pl.ds(0, CHUNK)], rows_v, sem).wait()
            pltpu.make_async_copy(base_hbm.at[pl.ds(0, CHUNK)], base_v, sem).wait()
            pltpu.make_async_copy(dst_hbm.at[pl.ds(0, CHUNK)], dst_v, sem).wait()
            pltpu.make_async_copy(src_hbm.at[pl.ds(0, CHUNK)], pf_src_v, sem).wait()

        def finish(c, rows_v, base_v, dst_v, pf_src_v, sem, sem_w):
            c = clamp(c)
            drain(rows_v, base_v, dst_v, pf_src_v, sem)
            @plsc.parallel_loop(0, CHUNK, unroll=4)
            def _row(r):
                for sl in range(DH // LANES):
                    col = pl.ds(sl * LANES, LANES)
                    rows_v[r, col] = jnp.maximum(rows_v[r, col] + base_v[r, col], 0.0)
            if write_h0:
                pltpu.async_copy(rows_v, h0_out.at[pl.ds(c * CHUNK, CHUNK)], sem_w)
            pltpu.sync_copy(rows_v, acc.at[dst_v], add=True)

        pltpu.sync_copy(src_hbm.at[pl.ds(clo * CHUNK, CHUNK)], src_a0)
        pltpu.sync_copy(src_hbm.at[pl.ds(clamp(clo + 1) * CHUNK, CHUNK)], src_b0)
        issue(clo, rows_a, base_a, dst_a, src_a0, clo + 2, src_a1, sem_a,
              sem_wa, False)

        def quad(qb, wd_b0):
            issue(qb + 1, rows_b, base_b, dst_b, src_b0, qb + 3, src_b1,
                  sem_b, sem_wb, wd_b0)
            finish(qb, rows_a, base_a, dst_a, src_a1, sem_a, sem_wa)
            issue(qb + 2, rows_a, base_a, dst_a, src_a1, qb + 4, src_a0,
                  sem_a, sem_wa, True)
            finish(qb + 1, rows_b, base_b, dst_b, src_b1, sem_b, sem_wb)
            issue(qb + 3, rows_b, base_b, dst_b, src_b1, qb + 5, src_b0,
                  sem_b, sem_wb, True)
            finish(qb + 2, rows_a, base_a, dst_a, src_a0, sem_a, sem_wa)
            issue(qb + 4, rows_a, base_a, dst_a, src_a0, qb + 6, src_a1,
                  sem_a, sem_wa, True)
            finish(qb + 3, rows_b, base_b, dst_b, src_b0, sem_b, sem_wb)

        quad(clo, False)

        def quad_loop(t, _):
            quad(clo + 4 * t, True)
            return _

        lax.fori_loop(1, nq, quad_loop, None)
        drain(rows_a, base_a, dst_a, src_a1, sem_a)
        if write_h0:
            pltpu.make_async_copy(base_hbm.at[pl.ds(0, CHUNK)], rows_b, sem_wb).wait()

        plsc.subcore_barrier()
        pltpu.sync_copy(acc.at[pl.ds(row0, RPT)], spart.at[cid, sid])

    return pl.kernel(body, out_type=tuple(out_type), mesh=mesh,
                     scratch_types=scratch_types)


_encoder_pass = _make_edge_pass(write_h0=True)
_layer_pass = _make_edge_pass(write_h0=False)


def _enc_body(a_ref, w2_ref, b_ref, x_ref, w1_ref, ea_ref, u_ref):
    ea_ref[...] = (jnp.dot(a_ref[...], w2_ref[...],
                           preferred_element_type=jnp.float32) + b_ref[...])
    @pl.when(pl.program_id(0) == 0)
    def _():
        u_ref[...] = jnp.dot(x_ref[...], w1_ref[...],
                             preferred_element_type=jnp.float32)


def _enc_kernel(edge_attr, w2, b2, x, w1):
    BE = 8000
    return pl.pallas_call(
        _enc_body,
        grid=(E // BE,),
        in_specs=[pl.BlockSpec((BE, DE), lambda i: (i, 0)),
                  pl.BlockSpec((DE, DH), lambda i: (0, 0)),
                  pl.BlockSpec((1, DH), lambda i: (0, 0)),
                  pl.BlockSpec((N, D), lambda i: (0, 0)),
                  pl.BlockSpec((D, DH), lambda i: (0, 0))],
        out_specs=[pl.BlockSpec((BE, DH), lambda i: (i, 0)),
                   pl.BlockSpec((N, DH), lambda i: (0, 0))],
        out_shape=[jax.ShapeDtypeStruct((E, DH), jnp.float32),
                   jax.ShapeDtypeStruct((N, DH), jnp.float32)],
    )(edge_attr, w2, b2, x, w1)


def _t_body(sp_ref, w_ref, b_ref, o_ref):
    s = sp_ref[0] + sp_ref[1]
    o_ref[...] = (jnp.dot(s, w_ref[...], preferred_element_type=jnp.float32)
                  + b_ref[...])


def _t_kernel(spart, w, b2):
    return pl.pallas_call(
        _t_body, out_shape=jax.ShapeDtypeStruct((N, DH), jnp.float32))(spart, w, b2)


def _final_body(x_ref, sp_ref, bv_ref, w1_ref, w2_ref, be_ref, wh_ref, bh_ref, o_ref):
    s = sp_ref[0] + sp_ref[1]
    xn = (jnp.dot(x_ref[...], w1_ref[...], preferred_element_type=jnp.float32)
          + jnp.dot(s, w2_ref[...], preferred_element_type=jnp.float32)
          + be_ref[...])
    xn = jnp.maximum(xn, 0.0)
    seg = lax.broadcasted_iota(jnp.int32, (N, G), 1)
    onehot = (bv_ref[...] == seg).astype(jnp.float32)
    pooled = lax.dot_general(onehot, xn, (((0,), (0,)), ((), ())),
                             preferred_element_type=jnp.float32)
    o_ref[...] = (jnp.dot(pooled, wh_ref[...], preferred_element_type=jnp.float32)
                  + bh_ref[...])


def _final_kernel(x, spart, bv2, w1, w2, be2, wh, bh2):
    return pl.pallas_call(
        _final_body, out_shape=jax.ShapeDtypeStruct((G, 1), jnp.float32),
    )(x, spart, bv2, w1, w2, be2, wh, bh2)


@jax.jit
def kernel(x, edge_index, edge_attr, batch_vec, W_enc, b_enc, W_layer, b_layer,
           W_e2n, b_e2n, W_head, b_head):
    src = edge_index[0].astype(jnp.int32)
    dst = edge_index[1].astype(jnp.int32)

    ea, u = _enc_kernel(edge_attr, W_enc[D:], b_enc.reshape(1, DH), x, W_enc[:D])

    h0, spart = _encoder_pass(ea, u, src, dst)
    spart = spart.reshape(NC, N, DH)
    b_layer2 = b_layer.reshape(1, DH)
    for _ in range(DEPTH):
        t = _t_kernel(spart, W_layer, b_layer2)
        (spart,) = _layer_pass(h0, t, src, dst)
        spart = spart.reshape(NC, N, DH)

    return _final_kernel(x, spart, batch_vec.astype(jnp.int32).reshape(N, 1),
                         W_e2n[:D], W_e2n[D:], b_e2n.reshape(1, DH),
                         W_head, b_head.reshape(1, 1))


# speedup vs baseline: 1.1475x; 1.0136x over previous
import functools

import jax
import jax.numpy as jnp
from jax import lax
from jax.experimental import pallas as pl
from jax.experimental.pallas import tpu as pltpu
from jax.experimental.pallas import tpu_sc as plsc

N = 10000
E = 320000
D = 128
DE = 16
DH = 128
G = 64
DEPTH = 3

NC = 2
NS = 16
NW = NC * NS
LANES = 16
CHUNK = 64
NCHUNKS = E // CHUNK
NQUADS = NCHUNKS // 4
RPT = N // NS


def _make_edge_pass(write_h0: bool):
    mesh = plsc.VectorSubcoreMesh(core_axis_name="c", subcore_axis_name="s")

    out_type = [jax.ShapeDtypeStruct((NC, NS, RPT, DH), jnp.float32)]
    if write_h0:
        out_type = [jax.ShapeDtypeStruct((E, DH), jnp.float32)] + out_type

    scratch_types = [
        pltpu.VMEM((CHUNK,), jnp.int32),
        pltpu.VMEM((CHUNK,), jnp.int32),
        pltpu.VMEM((CHUNK,), jnp.int32),
        pltpu.VMEM((CHUNK,), jnp.int32),
        pltpu.VMEM((CHUNK,), jnp.int32),
        pltpu.VMEM((CHUNK,), jnp.int32),
        pltpu.VMEM((CHUNK,), jnp.int32),
        pltpu.VMEM((CHUNK,), jnp.int32),
        pltpu.VMEM((CHUNK, DH), jnp.float32),
        pltpu.VMEM((CHUNK, DH), jnp.float32),
        pltpu.VMEM((CHUNK, DH), jnp.float32),
        pltpu.VMEM((CHUNK, DH), jnp.float32),
        pltpu.VMEM((CHUNK, DH), jnp.float32),
        pltpu.VMEM((CHUNK, DH), jnp.float32),
        pltpu.VMEM_SHARED((N, DH), jnp.float32),
        pltpu.SemaphoreType.DMA,
        pltpu.SemaphoreType.DMA,
        pltpu.SemaphoreType.DMA,
        pltpu.SemaphoreType.DMA,
        pltpu.SemaphoreType.DMA,
        pltpu.SemaphoreType.DMA,
    ]

    def body(base_hbm, table_hbm, src_hbm, dst_hbm, *rest):
        if write_h0:
            h0_out, spart, *rest = rest
        else:
            spart, *rest = rest
        (src_a0, src_a1, src_b0, src_b1, dst_a0, dst_a1, dst_b0, dst_b1,
         rows_a, rows_b, base_a, base_b, out_a, out_b, acc,
         sem_a, sem_b, sem_sa, sem_sb, sem_wa, sem_wb) = rest
        cid = lax.axis_index("c")
        sid = lax.axis_index("s")
        wid = sid * NC + cid

        def zrow(r, _):
            for sl in range(DH // LANES):
                rows_a[r, pl.ds(sl * LANES, LANES)] = jnp.zeros((LANES,), jnp.float32)
            return _
        lax.fori_loop(0, CHUNK, zrow, None)
        row0 = sid * RPT
        off = 0
        while off < RPT:
            nr = min(CHUNK, RPT - off)
            pltpu.sync_copy(rows_a.at[pl.ds(0, nr)], acc.at[pl.ds(row0 + off, nr)])
            off += nr
        plsc.subcore_barrier()

        qlo = (wid * NQUADS) // NW
        qhi = ((wid + 1) * NQUADS) // NW
        clo = 4 * qlo
        nq = qhi - qlo

        def clamp(c):
            return jnp.minimum(c, NCHUNKS - 1)

        def issue(c, rows_v, base_v, dst_v, src_v, pf_c, pf_src_v, sem):
            c = clamp(c)
            pltpu.async_copy(table_hbm.at[src_v], rows_v, sem)
            pltpu.async_copy(base_hbm.at[pl.ds(c * CHUNK, CHUNK)], base_v, sem)
            pltpu.async_copy(dst_hbm.at[pl.ds(c * CHUNK, CHUNK)], dst_v, sem)
            pltpu.async_copy(src_hbm.at[pl.ds(clamp(pf_c) * CHUNK, CHUNK)],
                             pf_src_v, sem)

        def drain(rows_v, base_v, dst_v, pf_src_v, sem):
            pltpu.make_async_copy(base_hbm.at[pl.ds(0, CHUNK)], rows_v, sem).wait()
            pltpu.make_async_copy(base_hbm.at[pl.ds(0, CHUNK)], base_v, sem).wait()
            pltpu.make_async_copy(dst_hbm.at[pl.ds(0, CHUNK)], dst_v, sem).wait()
            pltpu.make_async_copy(src_hbm.at[pl.ds(0, CHUNK)], pf_src_v, sem).wait()

        def compute(rows_v, base_v, out_v):
            @plsc.parallel_loop(0, CHUNK, unroll=4)
            def _row(r):
                for sl in range(DH // LANES):
                    col = pl.ds(sl * LANES, LANES)
                    out_v[r, col] = jnp.maximum(rows_v[r, col] + base_v[r, col], 0.0)

        def wdrain(out_v, sem_w):
            if write_h0:
                pltpu.make_async_copy(base_hbm.at[pl.ds(0, CHUNK)], out_v,
                                      sem_w).wait()

        def h0_write(c, out_v, sem_w):
            if write_h0:
                pltpu.async_copy(out_v, h0_out.at[pl.ds(clamp(c) * CHUNK, CHUNK)],
                                 sem_w)

        pltpu.sync_copy(src_hbm.at[pl.ds(clo * CHUNK, CHUNK)], src_a0)
        pltpu.sync_copy(src_hbm.at[pl.ds(clamp(clo + 1) * CHUNK, CHUNK)], src_b0)
        issue(clo, rows_a, base_a, dst_a0, src_a0, clo + 2, src_a1, sem_a)

        def quad(qb, first):
            issue(qb + 1, rows_b, base_b, dst_b0, src_b0, qb + 3, src_b1, sem_b)
            drain(rows_a, base_a, dst_a0, src_a1, sem_a)
            if not first:
                wdrain(out_a, sem_wa)
            compute(rows_a, base_a, out_a)
            h0_write(qb, out_a, sem_wa)
            d_a = pltpu.async_copy(out_a, acc.at[dst_a0], sem_sa, add=True)
            issue(qb + 2, rows_a, base_a, dst_a1, src_a1, qb + 4, src_a0, sem_a)
            drain(rows_b, base_b, dst_b0, src_b1, sem_b)
            if not first:
                wdrain(out_b, sem_wb)
            compute(rows_b, base_b, out_b)
            h0_write(qb + 1, out_b, sem_wb)
            d_b = pltpu.async_copy(out_b, acc.at[dst_b0], sem_sb, add=True)
            issue(qb + 3, rows_b, base_b, dst_b1, src_b1, qb + 5, src_b0, sem_b)
            drain(rows_a, base_a, dst_a1, src_a0, sem_a)
            d_a.wait()
            wdrain(out_a, sem_wa)
            compute(rows_a, base_a, out_a)
            h0_write(qb + 2, out_a, sem_wa)
            pltpu.sync_copy(out_a, acc.at[dst_a1], add=True)
            issue(qb + 4, rows_a, base_a, dst_a0, src_a0, qb + 6, src_a1, sem_a)
            drain(rows_b, base_b, dst_b1, src_b0, sem_b)
            d_b.wait()
            wdrain(out_b, sem_wb)
            compute(rows_b, base_b, out_b)
            h0_write(qb + 3, out_b, sem_wb)
            pltpu.sync_copy(out_b, acc.at[dst_b1], add=True)

        quad(clo, True)

        def quad_loop(t, _):
            quad(clo + 4 * t, False)
            return _

        lax.fori_loop(1, nq, quad_loop, None)
        drain(rows_a, base_a, dst_a0, src_a1, sem_a)
        wdrain(out_a, sem_wa)
        wdrain(out_b, sem_wb)

        plsc.subcore_barrier()
        pltpu.sync_copy(acc.at[pl.ds(row0, RPT)], spart.at[cid, sid])

    return pl.kernel(body, out_type=tuple(out_type), mesh=mesh,
                     scratch_types=scratch_types)


_encoder_pass = _make_edge_pass(write_h0=True)
_layer_pass = _make_edge_pass(write_h0=False)


def _enc_body(a_ref, w2_ref, b_ref, x_ref, w1_ref, ea_ref, u_ref):
    ea_ref[...] = (jnp.dot(a_ref[...], w2_ref[...],
                           preferred_element_type=jnp.float32) + b_ref[...])
    @pl.when(pl.program_id(0) == 0)
    def _():
        u_ref[...] = jnp.dot(x_ref[...], w1_ref[...],
                             preferred_element_type=jnp.float32)


def _enc_kernel(edge_attr, w2, b2, x, w1):
    BE = 8000
    return pl.pallas_call(
        _enc_body,
        grid=(E // BE,),
        in_specs=[pl.BlockSpec((BE, DE), lambda i: (i, 0)),
                  pl.BlockSpec((DE, DH), lambda i: (0, 0)),
                  pl.BlockSpec((1, DH), lambda i: (0, 0)),
                  pl.BlockSpec((N, D), lambda i: (0, 0)),
                  pl.BlockSpec((D, DH), lambda i: (0, 0))],
        out_specs=[pl.BlockSpec((BE, DH), lambda i: (i, 0)),
                   pl.BlockSpec((N, DH), lambda i: (0, 0))],
        out_shape=[jax.ShapeDtypeStruct((E, DH), jnp.float32),
                   jax.ShapeDtypeStruct((N, DH), jnp.float32)],
    )(edge_attr, w2, b2, x, w1)


def _t_body(sp_ref, w_ref, b_ref, o_ref):
    s = sp_ref[0] + sp_ref[1]
    o_ref[...] = (jnp.dot(s, w_ref[...], preferred_element_type=jnp.float32)
                  + b_ref[...])


def _t_kernel(spart, w, b2):
    return pl.pallas_call(
        _t_body, out_shape=jax.ShapeDtypeStruct((N, DH), jnp.float32))(spart, w, b2)


def _final_body(x_ref, sp_ref, bv_ref, w1_ref, w2_ref, be_ref, wh_ref, bh_ref, o_ref):
    s = sp_ref[0] + sp_ref[1]
    xn = (jnp.dot(x_ref[...], w1_ref[...], preferred_element_type=jnp.float32)
          + jnp.dot(s, w2_ref[...], preferred_element_type=jnp.float32)
          + be_ref[...])
    xn = jnp.maximum(xn, 0.0)
    seg = lax.broadcasted_iota(jnp.int32, (N, G), 1)
    onehot = (bv_ref[...] == seg).astype(jnp.float32)
    pooled = lax.dot_general(onehot, xn, (((0,), (0,)), ((), ())),
                             preferred_element_type=jnp.float32)
    o_ref[...] = (jnp.dot(pooled, wh_ref[...], preferred_element_type=jnp.float32)
                  + bh_ref[...])


def _final_kernel(x, spart, bv2, w1, w2, be2, wh, bh2):
    return pl.pallas_call(
        _final_body, out_shape=jax.ShapeDtypeStruct((G, 1), jnp.float32),
    )(x, spart, bv2, w1, w2, be2, wh, bh2)


@jax.jit
def kernel(x, edge_index, edge_attr, batch_vec, W_enc, b_enc, W_layer, b_layer,
           W_e2n, b_e2n, W_head, b_head):
    src = edge_index[0].astype(jnp.int32)
    dst = edge_index[1].astype(jnp.int32)

    ea, u = _enc_kernel(edge_attr, W_enc[D:], b_enc.reshape(1, DH), x, W_enc[:D])

    h0, spart = _encoder_pass(ea, u, src, dst)
    spart = spart.reshape(NC, N, DH)
    b_layer2 = b_layer.reshape(1, DH)
    for _ in range(DEPTH):
        t = _t_kernel(spart, W_layer, b_layer2)
        (spart,) = _layer_pass(h0, t, src, dst)
        spart = spart.reshape(NC, N, DH)

    return _final_kernel(x, spart, batch_vec.astype(jnp.int32).reshape(N, 1),
                         W_e2n[:D], W_e2n[D:], b_e2n.reshape(1, DH),
                         W_head, b_head.reshape(1, 1))


# speedup vs baseline: 1.1480x; 1.0004x over previous
import jax
import jax.numpy as jnp
from jax import lax
from jax.experimental import pallas as pl
from jax.experimental.pallas import tpu as pltpu
from jax.experimental.pallas import tpu_sc as plsc

N = 10000
E = 320000
D = 128
DE = 16
DH = 128
G = 64
DEPTH = 3

NC = 2
NS = 16
NW = NC * NS
LANES = 16
CHUNK = 64
NCHUNKS = E // CHUNK
NQUADS = NCHUNKS // 4
RPT = N // NS


def _make_edge_pass(write_h0: bool):
    mesh = plsc.VectorSubcoreMesh(core_axis_name="c", subcore_axis_name="s",
                                  num_cores=NC, num_subcores=NS)

    out_type = [jax.ShapeDtypeStruct((NC, NS, RPT, DH), jnp.float32)]
    if write_h0:
        out_type = [jax.ShapeDtypeStruct((E, DH), jnp.float32)] + out_type

    scratch_types = [
        pltpu.VMEM((CHUNK,), jnp.int32),
        pltpu.VMEM((CHUNK,), jnp.int32),
        pltpu.VMEM((CHUNK,), jnp.int32),
        pltpu.VMEM((CHUNK,), jnp.int32),
        pltpu.VMEM((CHUNK,), jnp.int32),
        pltpu.VMEM((CHUNK,), jnp.int32),
        pltpu.VMEM((CHUNK,), jnp.int32),
        pltpu.VMEM((CHUNK,), jnp.int32),
        pltpu.VMEM((CHUNK, DH), jnp.float32),
        pltpu.VMEM((CHUNK, DH), jnp.float32),
        pltpu.VMEM((CHUNK, DH), jnp.float32),
        pltpu.VMEM((CHUNK, DH), jnp.float32),
        pltpu.VMEM((CHUNK, DH), jnp.float32),
        pltpu.VMEM((CHUNK, DH), jnp.float32),
        pltpu.VMEM_SHARED((N, DH), jnp.float32),
        pltpu.SemaphoreType.DMA,
        pltpu.SemaphoreType.DMA,
        pltpu.SemaphoreType.DMA,
        pltpu.SemaphoreType.DMA,
        pltpu.SemaphoreType.DMA,
        pltpu.SemaphoreType.DMA,
    ]

    def body(base_hbm, table_hbm, src_hbm, dst_hbm, *rest):
        if write_h0:
            h0_out, spart, *rest = rest
        else:
            spart, *rest = rest
        (src_a0, src_a1, src_b0, src_b1, dst_a0, dst_a1, dst_b0, dst_b1,
         rows_a, rows_b, base_a, base_b, out_a, out_b, acc,
         sem_a, sem_b, sem_sa, sem_sb, sem_wa, sem_wb) = rest
        cid = lax.axis_index("c")
        sid = lax.axis_index("s")
        wid = sid * NC + cid

        def zrow(r, _):
            for sl in range(DH // LANES):
                rows_a[r, pl.ds(sl * LANES, LANES)] = jnp.zeros((LANES,), jnp.float32)
            return _
        lax.fori_loop(0, CHUNK, zrow, None)
        row0 = sid * RPT
        off = 0
        while off < RPT:
            nr = min(CHUNK, RPT - off)
            pltpu.sync_copy(rows_a.at[pl.ds(0, nr)], acc.at[pl.ds(row0 + off, nr)])
            off += nr
        plsc.subcore_barrier()

        qlo = (wid * NQUADS) // NW
        qhi = ((wid + 1) * NQUADS) // NW
        clo = 4 * qlo
        nq = qhi - qlo

        def clamp(c):
            return jnp.minimum(c, NCHUNKS - 1)

        def issue(c, rows_v, base_v, dst_v, src_v, pf_c, pf_src_v, sem):
            c = clamp(c)
            pltpu.async_copy(table_hbm.at[src_v], rows_v, sem)
            pltpu.async_copy(base_hbm.at[pl.ds(c * CHUNK, CHUNK)], base_v, sem)
            pltpu.async_copy(dst_hbm.at[pl.ds(c * CHUNK, CHUNK)], dst_v, sem)
            pltpu.async_copy(src_hbm.at[pl.ds(clamp(pf_c) * CHUNK, CHUNK)],
                             pf_src_v, sem)

        def drain(rows_v, base_v, dst_v, pf_src_v, sem):
            pltpu.make_async_copy(base_hbm.at[pl.ds(0, CHUNK)], rows_v, sem).wait()
            pltpu.make_async_copy(base_hbm.at[pl.ds(0, CHUNK)], base_v, sem).wait()
            pltpu.make_async_copy(dst_hbm.at[pl.ds(0, CHUNK)], dst_v, sem).wait()
            pltpu.make_async_copy(src_hbm.at[pl.ds(0, CHUNK)], pf_src_v, sem).wait()

        def compute(rows_v, base_v, out_v):
            @plsc.parallel_loop(0, CHUNK, unroll=4)
            def _row(r):
                for sl in range(DH // LANES):
                    col = pl.ds(sl * LANES, LANES)
                    out_v[r, col] = jnp.maximum(rows_v[r, col] + base_v[r, col], 0.0)

        def wdrain(out_v, sem_w):
            if write_h0:
                pltpu.make_async_copy(base_hbm.at[pl.ds(0, CHUNK)], out_v,
                                      sem_w).wait()

        def h0_write(c, out_v, sem_w):
            if write_h0:
                pltpu.async_copy(out_v, h0_out.at[pl.ds(clamp(c) * CHUNK, CHUNK)],
                                 sem_w)

        pltpu.sync_copy(src_hbm.at[pl.ds(clo * CHUNK, CHUNK)], src_a0)
        pltpu.sync_copy(src_hbm.at[pl.ds(clamp(clo + 1) * CHUNK, CHUNK)], src_b0)
        issue(clo, rows_a, base_a, dst_a0, src_a0, clo + 2, src_a1, sem_a)

        def quad(qb, first):
            issue(qb + 1, rows_b, base_b, dst_b0, src_b0, qb + 3, src_b1, sem_b)
            drain(rows_a, base_a, dst_a0, src_a1, sem_a)
            if not first:
                wdrain(out_a, sem_wa)
            compute(rows_a, base_a, out_a)
            h0_write(qb, out_a, sem_wa)
            d_a = pltpu.async_copy(out_a, acc.at[dst_a0], sem_sa, add=True)
            issue(qb + 2, rows_a, base_a, dst_a1, src_a1, qb + 4, src_a0, sem_a)
            drain(rows_b, base_b, dst_b0, src_b1, sem_b)
            if not first:
                wdrain(out_b, sem_wb)
            compute(rows_b, base_b, out_b)
            h0_write(qb + 1, out_b, sem_wb)
            d_b = pltpu.async_copy(out_b, acc.at[dst_b0], sem_sb, add=True)
            issue(qb + 3, rows_b, base_b, dst_b1, src_b1, qb + 5, src_b0, sem_b)
            drain(rows_a, base_a, dst_a1, src_a0, sem_a)
            d_a.wait()
            wdrain(out_a, sem_wa)
            compute(rows_a, base_a, out_a)
            h0_write(qb + 2, out_a, sem_wa)
            pltpu.sync_copy(out_a, acc.at[dst_a1], add=True)
            issue(qb + 4, rows_a, base_a, dst_a0, src_a0, qb + 6, src_a1, sem_a)
            drain(rows_b, base_b, dst_b1, src_b0, sem_b)
            d_b.wait()
            wdrain(out_b, sem_wb)
            compute(rows_b, base_b, out_b)
            h0_write(qb + 3, out_b, sem_wb)
            pltpu.sync_copy(out_b, acc.at[dst_b1], add=True)

        quad(clo, True)

        def quad_loop(t, _):
            quad(clo + 4 * t, False)
            return _

        lax.fori_loop(1, nq, quad_loop, None)
        drain(rows_a, base_a, dst_a0, src_a1, sem_a)
        wdrain(out_a, sem_wa)
        wdrain(out_b, sem_wb)

        plsc.subcore_barrier()
        pltpu.sync_copy(acc.at[pl.ds(row0, RPT)], spart.at[cid, sid])

    return pl.kernel(body, out_type=tuple(out_type), mesh=mesh,
                     scratch_types=scratch_types)


_encoder_pass = _make_edge_pass(write_h0=True)
_layer_pass = _make_edge_pass(write_h0=False)


def _enc_body(a_ref, w2_ref, b_ref, x_ref, w1_ref, ea_ref, u_ref):
    ea_ref[...] = (jnp.dot(a_ref[...], w2_ref[...],
                           preferred_element_type=jnp.float32) + b_ref[...])
    @pl.when(pl.program_id(0) == 0)
    def _():
        u_ref[...] = jnp.dot(x_ref[...], w1_ref[...],
                             preferred_element_type=jnp.float32)


def _enc_kernel(edge_attr, w2, b2, x, w1):
    BE = 8000
    return pl.pallas_call(
        _enc_body,
        grid=(E // BE,),
        in_specs=[pl.BlockSpec((BE, DE), lambda i: (i, 0)),
                  pl.BlockSpec((DE, DH), lambda i: (0, 0)),
                  pl.BlockSpec((1, DH), lambda i: (0, 0)),
                  pl.BlockSpec((N, D), lambda i: (0, 0)),
                  pl.BlockSpec((D, DH), lambda i: (0, 0))],
        out_specs=[pl.BlockSpec((BE, DH), lambda i: (i, 0)),
                   pl.BlockSpec((N, DH), lambda i: (0, 0))],
        out_shape=[jax.ShapeDtypeStruct((E, DH), jnp.float32),
                   jax.ShapeDtypeStruct((N, DH), jnp.float32)],
    )(edge_attr, w2, b2, x, w1)


def _t_body(sp_ref, w_ref, b_ref, o_ref):
    s = sp_ref[0] + sp_ref[1]
    o_ref[...] = (jnp.dot(s, w_ref[...], preferred_element_type=jnp.float32)
                  + b_ref[...])


def _t_kernel(spart, w, b2):
    return pl.pallas_call(
        _t_body, out_shape=jax.ShapeDtypeStruct((N, DH), jnp.float32))(spart, w, b2)


def _final_body(x_ref, sp_ref, bv_ref, w1_ref, w2_ref, be_ref, wh_ref, bh_ref, o_ref):
    s = sp_ref[0] + sp_ref[1]
    xn = (jnp.dot(x_ref[...], w1_ref[...], preferred_element_type=jnp.float32)
          + jnp.dot(s, w2_ref[...], preferred_element_type=jnp.float32)
          + be_ref[...])
    xn = jnp.maximum(xn, 0.0)
    seg = lax.broadcasted_iota(jnp.int32, (N, G), 1)
    onehot = (bv_ref[...] == seg).astype(jnp.float32)
    pooled = lax.dot_general(onehot, xn, (((0,), (0,)), ((), ())),
                             preferred_element_type=jnp.float32)
    o_ref[...] = (jnp.dot(pooled, wh_ref[...], preferred_element_type=jnp.float32)
                  + bh_ref[...])


def _final_kernel(x, spart, bv2, w1, w2, be2, wh, bh2):
    return pl.pallas_call(
        _final_body, out_shape=jax.ShapeDtypeStruct((G, 1), jnp.float32),
    )(x, spart, bv2, w1, w2, be2, wh, bh2)


@jax.jit
def kernel(x, edge_index, edge_attr, batch_vec, W_enc, b_enc, W_layer, b_layer,
           W_e2n, b_e2n, W_head, b_head):
    src = edge_index[0].astype(jnp.int32)
    dst = edge_index[1].astype(jnp.int32)

    ea, u = _enc_kernel(edge_attr, W_enc[D:], b_enc.reshape(1, DH), x, W_enc[:D])

    h0, spart = _encoder_pass(ea, u, src, dst)
    spart = spart.reshape(NC, N, DH)
    b_layer2 = b_layer.reshape(1, DH)
    for _ in range(DEPTH):
        t = _t_kernel(spart, W_layer, b_layer2)
        (spart,) = _layer_pass(h0, t, src, dst)
        spart = spart.reshape(NC, N, DH)

    return _final_kernel(x, spart, batch_vec.astype(jnp.int32).reshape(N, 1),
                         W_e2n[:D], W_e2n[D:], b_e2n.reshape(1, DH),
                         W_head, b_head.reshape(1, 1))
